# baseline (speedup 1.0000x reference)
import dataclasses
import functools

import jax
import jax.numpy as jnp
from jax import lax
from jax.experimental import pallas as pl
from jax.experimental.pallas import tpu as pltpu
from jax.experimental.pallas import tpu_sc as plsc

N = 10000
E = 320000
D = 128
H = 128
C = 10
B = 128

NC = 2
NS = 16
L = 16
NW = NC * NS
K = 128
EP = 327680
EPT = EP // NW
NCHUNK = EPT // K
NBUF = 2
NP = 10240
TCHUNK = 2 * NCHUNK
CH0 = 120
CH1 = TCHUNK - CH0
RPT = NP // NS
ZR = 128

_mesh = plsc.VectorSubcoreMesh(core_axis_name="c", subcore_axis_name="s")

_sc_params = pltpu.CompilerParams()
if "needs_layout_passes" in pltpu.CompilerParams.__dataclass_fields__:
    _sc_params = dataclasses.replace(_sc_params, needs_layout_passes=False)


@functools.partial(
    pl.kernel,
    out_type=jax.ShapeDtypeStruct((NC, NP), jnp.float32),
    mesh=_mesh,
    compiler_params=_sc_params,
    scratch_types=[
        pltpu.VMEM((NCHUNK, K), jnp.int32),
        pltpu.VMEM((NCHUNK, K), jnp.float32),
        pltpu.VMEM((RPT,), jnp.float32),
        pltpu.VMEM_SHARED((NP,), jnp.float32),
    ],
)
def _deg_kernel(col_hbm, ew_hbm, out_hbm, col_v, ew_v, zero_v, acc_sh):
    cid = lax.axis_index("c")
    sid = lax.axis_index("s")
    wid = sid * NC + cid

    pltpu.sync_copy(col_hbm.at[wid], col_v)
    pltpu.sync_copy(ew_hbm.at[wid], ew_v)

    zvec = jnp.zeros((L,), jnp.float32)

    @pl.loop(0, RPT, step=L)
    def _(r):
        zero_v[pl.ds(r, L)] = zvec

    pltpu.sync_copy(zero_v, acc_sh.at[pl.ds(sid * RPT, RPT)])
    plsc.subcore_barrier()

    @pl.loop(0, NCHUNK)
    def _(j):
        pltpu.sync_copy(ew_v.at[j], acc_sh.at[col_v.at[j]], add=True)

    plsc.subcore_barrier()
    pltpu.sync_copy(acc_sh.at[pl.ds(sid * RPT, RPT)],
                    out_hbm.at[cid, pl.ds(sid * RPT, RPT)])


CPP = 8


@functools.partial(
    pl.kernel,
    out_type=jax.ShapeDtypeStruct((NC, NP, D), jnp.float32),
    mesh=_mesh,
    compiler_params=_sc_params,
    scratch_types=[
        pltpu.VMEM((CPP, K), jnp.int32),
        pltpu.VMEM((CPP, K), jnp.int32),
        pltpu.VMEM((CPP, K), jnp.float32),
        pltpu.VMEM((NBUF, K, D), jnp.float32),
        pltpu.VMEM_SHARED((NP, D), jnp.float32),
    ] + [pltpu.SemaphoreType.DMA] * NBUF,
)
def _msg_kernel(y_hbm, row_hbm, col_hbm, ew_hbm, out_hbm,
                row_v, col_v, ew_v, bufs, acc_sh, *sems):
    cid = lax.axis_index("c")
    sid = lax.axis_index("s")
    wid = sid * NC + cid

    zvec = jnp.zeros((L,), jnp.float32)

    @pl.loop(0, K)
    def _(r):
        for q in range(0, D, L):
            bufs[0, r, pl.ds(q, L)] = zvec

    @pl.loop(0, RPT, step=K)
    def _(r0):
        pltpu.sync_copy(bufs.at[0], acc_sh.at[pl.ds(sid * RPT + r0, K)])

    plsc.subcore_barrier()

    def _do_chunk(j, b, sem):
        buf = bufs.at[b]
        pltpu.make_async_copy(y_hbm.at[row_v.at[j]], buf, sem).wait()

        @pl.loop(0, K, step=2)
        def _(e):
            jv = jnp.full((L,), j, jnp.int32)
            for u in range(2):
                s = plsc.load_gather(ew_v, [jv, jnp.full((L,), e + u, jnp.int32)])
                for q in range(0, D, L):
                    bufs[b, e + u, pl.ds(q, L)] = bufs[b, e + u, pl.ds(q, L)] * s

        pltpu.sync_copy(buf, acc_sh.at[col_v.at[j]], add=True)

        @pl.when(j + NBUF < CPP)
        def _():
            pltpu.async_copy(y_hbm.at[row_v.at[j + NBUF]], buf, sem)

    def _run_phases(nph, chunk_base):
        @pl.loop(0, nph)
        def _(p):
            c0 = chunk_base + p * CPP
            pltpu.sync_copy(row_hbm.at[sid, pl.ds(c0, CPP)], row_v)
            pltpu.sync_copy(col_hbm.at[sid, pl.ds(c0, CPP)], col_v)
            pltpu.sync_copy(ew_hbm.at[sid, pl.ds(c0, CPP)], ew_v)

            for b in range(NBUF):
                pltpu.async_copy(y_hbm.at[row_v.at[b]], bufs.at[b], sems[b])

            @pl.loop(0, CPP, step=NBUF)
            def _(j):
                for b in range(NBUF):
                    _do_chunk(j + b, b, sems[b])

    @pl.when(cid == 0)
    def _():
        _run_phases(CH0 // CPP, 0)

    @pl.when(cid == 1)
    def _():
        _run_phases(CH1 // CPP, CH0)

    plsc.subcore_barrier()

    @pl.loop(0, RPT, step=ZR)
    def _(r0):
        r = sid * RPT + r0
        pltpu.sync_copy(acc_sh.at[pl.ds(r, ZR)], out_hbm.at[cid, pl.ds(r, ZR)])


BLK = 400
GRID = N // BLK


def _mm_scale_body(x_ref, w_ref, degp_ref, y_ref):
    xw = jnp.dot(x_ref[...], w_ref[...], preferred_element_type=jnp.float32)
    y_ref[...] = xw * _dinv_of(degp_ref)


def _mm_scale(x, w, degp):
    return pl.pallas_call(
        _mm_scale_body,
        grid=(GRID,),
        in_specs=[pl.BlockSpec((BLK, D), lambda i: (i, 0)),
                  pl.BlockSpec((D, H), lambda i: (0, 0)),
                  pl.BlockSpec((BLK, NC), lambda i: (i, 0))],
        out_specs=pl.BlockSpec((BLK, H), lambda i: (i, 0)),
        out_shape=jax.ShapeDtypeStruct((N, H), jnp.float32),
    )(x, w, degp)


def _dinv_of(degp_ref):
    deg = degp_ref[:, 0:1] + degp_ref[:, 1:2] + 1.0
    return lax.rsqrt(deg)


def _combine_body(s_ref, degp_ref, y_ref, b_ref, w_ref, y2_ref):
    dinv = _dinv_of(degp_ref)
    h = (s_ref[0] + s_ref[1] + y_ref[...]) * dinv + b_ref[...]
    h = jnp.maximum(h, 0.0)
    y2_ref[...] = jnp.dot(h, w_ref[...], preferred_element_type=jnp.float32) * dinv


def _combine(s, degp, y, b, w):
    return pl.pallas_call(
        _combine_body,
        grid=(GRID,),
        in_specs=[pl.BlockSpec((NC, BLK, D), lambda i: (0, i, 0)),
                  pl.BlockSpec((BLK, NC), lambda i: (i, 0)),
                  pl.BlockSpec((BLK, H), lambda i: (i, 0)),
                  pl.BlockSpec((1, H), lambda i: (0, 0)),
                  pl.BlockSpec((H, H), lambda i: (0, 0))],
        out_specs=pl.BlockSpec((BLK, H), lambda i: (i, 0)),
        out_shape=jax.ShapeDtypeStruct((N, H), jnp.float32),
    )(s, degp, y, b.reshape(1, H), w)


def _final_body(s_ref, degp_ref, y_ref, b_ref, bid_ref,
                l1w_ref, l1b_ref, lng_ref, lnb_ref, l2w_ref, l2b_ref,
                out_ref, sums_ref, cnts_ref):
    i = pl.program_id(0)

    @pl.when(i == 0)
    def _():
        sums_ref[...] = jnp.zeros_like(sums_ref)
        cnts_ref[...] = jnp.zeros_like(cnts_ref)

    dinv = _dinv_of(degp_ref)
    h = (s_ref[0] + s_ref[1] + y_ref[...]) * dinv + b_ref[...]
    bid = bid_ref[0, 0, :]
    onehot = (lax.broadcasted_iota(jnp.int32, (B, BLK), 0) == bid[None, :]
              ).astype(jnp.float32)
    sums_ref[...] += jnp.dot(onehot, h, preferred_element_type=jnp.float32)
    cnts_ref[...] += jnp.sum(onehot, axis=1, keepdims=True)

    @pl.when(i == GRID - 1)
    def _():
        g = sums_ref[...] / jnp.maximum(cnts_ref[:, 0:1], 1.0)
        z = jnp.dot(g, l1w_ref[...], preferred_element_type=jnp.float32) + l1b_ref[...]
        z = jnp.maximum(z, 0.0)
        mu = jnp.mean(z, axis=-1, keepdims=True)
        var = jnp.mean((z - mu) ** 2, axis=-1, keepdims=True)
        zn = (z - mu) * lax.rsqrt(var + 1e-5) * lng_ref[...] + lnb_ref[...]
        out_ref[...] = (jnp.dot(zn, l2w_ref[...], preferred_element_type=jnp.float32)
                        + l2b_ref[...])


def _final(s, degp, y, b, batch_idx, l1w, l1b, lng, lnb, l2w, l2b):
    return pl.pallas_call(
        _final_body,
        grid=(GRID,),
        in_specs=[pl.BlockSpec((NC, BLK, D), lambda i: (0, i, 0)),
                  pl.BlockSpec((BLK, NC), lambda i: (i, 0)),
                  pl.BlockSpec((BLK, H), lambda i: (i, 0)),
                  pl.BlockSpec((1, H), lambda i: (0, 0)),
                  pl.BlockSpec((1, 1, BLK), lambda i: (i, 0, 0)),
                  pl.BlockSpec((H, H), lambda i: (0, 0)),
                  pl.BlockSpec((1, H), lambda i: (0, 0)),
                  pl.BlockSpec((1, H), lambda i: (0, 0)),
                  pl.BlockSpec((1, H), lambda i: (0, 0)),
                  pl.BlockSpec((H, C), lambda i: (0, 0)),
                  pl.BlockSpec((1, C), lambda i: (0, 0))],
        out_specs=pl.BlockSpec((B, C), lambda i: (0, 0)),
        out_shape=jax.ShapeDtypeStruct((B, C), jnp.float32),
        scratch_shapes=[pltpu.VMEM((B, H), jnp.float32),
                        pltpu.VMEM((B, H), jnp.float32)],
    )(s, degp, y, b.reshape(1, H), batch_idx.reshape(GRID, 1, BLK),
      l1w, l1b.reshape(1, H), lng.reshape(1, H), lnb.reshape(1, H),
      l2w, l2b.reshape(1, C))


def kernel(x, edge_index, edge_weight, batch_idx,
           W1, b1, W2, b2, L1W, L1b, ln_g, ln_b, L2W, L2b):
    pad = EP - E
    row = jnp.pad(edge_index[0], (0, pad)).reshape(NS, TCHUNK, K)
    col = jnp.pad(edge_index[1], (0, pad)).reshape(NS, TCHUNK, K)
    ew2 = jnp.pad(edge_weight, (0, pad)).reshape(NS, TCHUNK, K)
    degp = _deg_kernel(col.reshape(NW, NCHUNK, K), ew2.reshape(NW, NCHUNK, K)).T
    y1 = _mm_scale(x, W1, degp)
    s1 = _msg_kernel(y1, row, col, ew2)
    y2 = _combine(s1, degp, y1, b1, W2)
    s2 = _msg_kernel(y2, row, col, ew2)
    return _final(s2, degp, y2, b2, batch_idx, L1W, L1b, ln_g, ln_b, L2W, L2b)


# speedup vs baseline: 1.0641x; 1.0641x over previous
import dataclasses
import functools

import jax
import jax.numpy as jnp
from jax import lax
from jax.experimental import pallas as pl
from jax.experimental.pallas import tpu as pltpu
from jax.experimental.pallas import tpu_sc as plsc

N = 10000
E = 320000
D = 128
H = 128
C = 10
B = 128

NC = 2
NS = 16
L = 16
NW = NC * NS
K = 128
EP = 327680
EPT = EP // NW
NCHUNK = EPT // K
NBUF = 2
NP = 10240
TCHUNK = 2 * NCHUNK
CH0 = 120
CH1 = TCHUNK - CH0
RPT = NP // NS
ZR = 128

_mesh = plsc.VectorSubcoreMesh(core_axis_name="c", subcore_axis_name="s")

_sc_params = pltpu.CompilerParams()
if "needs_layout_passes" in pltpu.CompilerParams.__dataclass_fields__:
    _sc_params = dataclasses.replace(_sc_params, needs_layout_passes=False)


@functools.partial(
    pl.kernel,
    out_type=jax.ShapeDtypeStruct((NC, NP), jnp.float32),
    mesh=_mesh,
    compiler_params=_sc_params,
    scratch_types=[
        pltpu.VMEM((NCHUNK, K), jnp.int32),
        pltpu.VMEM((NCHUNK, K), jnp.float32),
        pltpu.VMEM((RPT,), jnp.float32),
        pltpu.VMEM_SHARED((NP,), jnp.float32),
    ],
)
def _deg_kernel(col_hbm, ew_hbm, out_hbm, col_v, ew_v, zero_v, acc_sh):
    cid = lax.axis_index("c")
    sid = lax.axis_index("s")
    wid = sid * NC + cid

    pltpu.sync_copy(col_hbm.at[wid], col_v)
    pltpu.sync_copy(ew_hbm.at[wid], ew_v)

    zvec = jnp.zeros((L,), jnp.float32)

    @pl.loop(0, RPT, step=L)
    def _(r):
        zero_v[pl.ds(r, L)] = zvec

    pltpu.sync_copy(zero_v, acc_sh.at[pl.ds(sid * RPT, RPT)])
    plsc.subcore_barrier()

    @pl.loop(0, NCHUNK)
    def _(j):
        pltpu.sync_copy(ew_v.at[j], acc_sh.at[col_v.at[j]], add=True)

    plsc.subcore_barrier()
    pltpu.sync_copy(acc_sh.at[pl.ds(sid * RPT, RPT)],
                    out_hbm.at[cid, pl.ds(sid * RPT, RPT)])


CPP = 40


@functools.partial(
    pl.kernel,
    out_type=jax.ShapeDtypeStruct((NC, NP, D), jnp.float32),
    mesh=_mesh,
    compiler_params=_sc_params,
    scratch_types=[
        pltpu.VMEM((CPP, K), jnp.int32),
        pltpu.VMEM((CPP, K), jnp.int32),
        pltpu.VMEM((CPP, K), jnp.float32),
        pltpu.VMEM((NBUF, K, D), jnp.float32),
        pltpu.VMEM_SHARED((NP, D), jnp.float32),
    ] + [pltpu.SemaphoreType.DMA] * NBUF,
)
def _msg_kernel(y_hbm, row_hbm, col_hbm, ew_hbm, out_hbm,
                row_v, col_v, ew_v, bufs, acc_sh, *sems):
    cid = lax.axis_index("c")
    sid = lax.axis_index("s")
    wid = sid * NC + cid

    zvec = jnp.zeros((L,), jnp.float32)

    @pl.loop(0, K)
    def _(r):
        for q in range(0, D, L):
            bufs[0, r, pl.ds(q, L)] = zvec

    @pl.loop(0, RPT, step=K)
    def _(r0):
        pltpu.sync_copy(bufs.at[0], acc_sh.at[pl.ds(sid * RPT + r0, K)])

    plsc.subcore_barrier()

    def _do_chunk(j, b, sem):
        buf = bufs.at[b]
        pltpu.make_async_copy(y_hbm.at[row_v.at[j]], buf, sem).wait()

        @pl.loop(0, K, step=2)
        def _(e):
            jv = jnp.full((L,), j, jnp.int32)
            for u in range(2):
                s = plsc.load_gather(ew_v, [jv, jnp.full((L,), e + u, jnp.int32)])
                for q in range(0, D, L):
                    bufs[b, e + u, pl.ds(q, L)] = bufs[b, e + u, pl.ds(q, L)] * s

        pltpu.sync_copy(buf, acc_sh.at[col_v.at[j]], add=True)

        @pl.when(j + NBUF < CPP)
        def _():
            pltpu.async_copy(y_hbm.at[row_v.at[j + NBUF]], buf, sem)

    def _run_phases(nph, chunk_base):
        @pl.loop(0, nph)
        def _(p):
            c0 = chunk_base + p * CPP
            pltpu.sync_copy(row_hbm.at[sid, pl.ds(c0, CPP)], row_v)
            pltpu.sync_copy(col_hbm.at[sid, pl.ds(c0, CPP)], col_v)
            pltpu.sync_copy(ew_hbm.at[sid, pl.ds(c0, CPP)], ew_v)

            for b in range(NBUF):
                pltpu.async_copy(y_hbm.at[row_v.at[b]], bufs.at[b], sems[b])

            @pl.loop(0, CPP, step=NBUF)
            def _(j):
                for b in range(NBUF):
                    _do_chunk(j + b, b, sems[b])

    @pl.when(cid == 0)
    def _():
        _run_phases(CH0 // CPP, 0)

    @pl.when(cid == 1)
    def _():
        _run_phases(CH1 // CPP, CH0)

    plsc.subcore_barrier()

    @pl.loop(0, RPT, step=ZR)
    def _(r0):
        r = sid * RPT + r0
        pltpu.sync_copy(acc_sh.at[pl.ds(r, ZR)], out_hbm.at[cid, pl.ds(r, ZR)])


BLK = 400
GRID = N // BLK


def _mm_scale_body(x_ref, w_ref, degp_ref, y_ref):
    xw = jnp.dot(x_ref[...], w_ref[...], preferred_element_type=jnp.float32)
    y_ref[...] = xw * _dinv_of(degp_ref)


def _mm_scale(x, w, degp):
    return pl.pallas_call(
        _mm_scale_body,
        grid=(GRID,),
        in_specs=[pl.BlockSpec((BLK, D), lambda i: (i, 0)),
                  pl.BlockSpec((D, H), lambda i: (0, 0)),
                  pl.BlockSpec((BLK, NC), lambda i: (i, 0))],
        out_specs=pl.BlockSpec((BLK, H), lambda i: (i, 0)),
        out_shape=jax.ShapeDtypeStruct((N, H), jnp.float32),
    )(x, w, degp)


def _dinv_of(degp_ref):
    deg = degp_ref[:, 0:1] + degp_ref[:, 1:2] + 1.0
    return lax.rsqrt(deg)


def _combine_body(s_ref, degp_ref, y_ref, b_ref, w_ref, y2_ref):
    dinv = _dinv_of(degp_ref)
    h = (s_ref[0] + s_ref[1] + y_ref[...]) * dinv + b_ref[...]
    h = jnp.maximum(h, 0.0)
    y2_ref[...] = jnp.dot(h, w_ref[...], preferred_element_type=jnp.float32) * dinv


def _combine(s, degp, y, b, w):
    return pl.pallas_call(
        _combine_body,
        grid=(GRID,),
        in_specs=[pl.BlockSpec((NC, BLK, D), lambda i: (0, i, 0)),
                  pl.BlockSpec((BLK, NC), lambda i: (i, 0)),
                  pl.BlockSpec((BLK, H), lambda i: (i, 0)),
                  pl.BlockSpec((1, H), lambda i: (0, 0)),
                  pl.BlockSpec((H, H), lambda i: (0, 0))],
        out_specs=pl.BlockSpec((BLK, H), lambda i: (i, 0)),
        out_shape=jax.ShapeDtypeStruct((N, H), jnp.float32),
    )(s, degp, y, b.reshape(1, H), w)


def _final_body(s_ref, degp_ref, y_ref, b_ref, bid_ref,
                l1w_ref, l1b_ref, lng_ref, lnb_ref, l2w_ref, l2b_ref,
                out_ref, sums_ref, cnts_ref):
    i = pl.program_id(0)

    @pl.when(i == 0)
    def _():
        sums_ref[...] = jnp.zeros_like(sums_ref)
        cnts_ref[...] = jnp.zeros_like(cnts_ref)

    dinv = _dinv_of(degp_ref)
    h = (s_ref[0] + s_ref[1] + y_ref[...]) * dinv + b_ref[...]
    bid = bid_ref[0, 0, :]
    onehot = (lax.broadcasted_iota(jnp.int32, (B, BLK), 0) == bid[None, :]
              ).astype(jnp.float32)
    sums_ref[...] += jnp.dot(onehot, h, preferred_element_type=jnp.float32)
    cnts_ref[...] += jnp.sum(onehot, axis=1, keepdims=True)

    @pl.when(i == GRID - 1)
    def _():
        g = sums_ref[...] / jnp.maximum(cnts_ref[:, 0:1], 1.0)
        z = jnp.dot(g, l1w_ref[...], preferred_element_type=jnp.float32) + l1b_ref[...]
        z = jnp.maximum(z, 0.0)
        mu = jnp.mean(z, axis=-1, keepdims=True)
        var = jnp.mean((z - mu) ** 2, axis=-1, keepdims=True)
        zn = (z - mu) * lax.rsqrt(var + 1e-5) * lng_ref[...] + lnb_ref[...]
        out_ref[...] = (jnp.dot(zn, l2w_ref[...], preferred_element_type=jnp.float32)
                        + l2b_ref[...])


def _final(s, degp, y, b, batch_idx, l1w, l1b, lng, lnb, l2w, l2b):
    return pl.pallas_call(
        _final_body,
        grid=(GRID,),
        in_specs=[pl.BlockSpec((NC, BLK, D), lambda i: (0, i, 0)),
                  pl.BlockSpec((BLK, NC), lambda i: (i, 0)),
                  pl.BlockSpec((BLK, H), lambda i: (i, 0)),
                  pl.BlockSpec((1, H), lambda i: (0, 0)),
                  pl.BlockSpec((1, 1, BLK), lambda i: (i, 0, 0)),
                  pl.BlockSpec((H, H), lambda i: (0, 0)),
                  pl.BlockSpec((1, H), lambda i: (0, 0)),
                  pl.BlockSpec((1, H), lambda i: (0, 0)),
                  pl.BlockSpec((1, H), lambda i: (0, 0)),
                  pl.BlockSpec((H, C), lambda i: (0, 0)),
                  pl.BlockSpec((1, C), lambda i: (0, 0))],
        out_specs=pl.BlockSpec((B, C), lambda i: (0, 0)),
        out_shape=jax.ShapeDtypeStruct((B, C), jnp.float32),
        scratch_shapes=[pltpu.VMEM((B, H), jnp.float32),
                        pltpu.VMEM((B, H), jnp.float32)],
    )(s, degp, y, b.reshape(1, H), batch_idx.reshape(GRID, 1, BLK),
      l1w, l1b.reshape(1, H), lng.reshape(1, H), lnb.reshape(1, H),
      l2w, l2b.reshape(1, C))


def kernel(x, edge_index, edge_weight, batch_idx,
           W1, b1, W2, b2, L1W, L1b, ln_g, ln_b, L2W, L2b):
    pad = EP - E
    row = jnp.pad(edge_index[0], (0, pad)).reshape(NS, TCHUNK, K)
    col = jnp.pad(edge_index[1], (0, pad)).reshape(NS, TCHUNK, K)
    ew2 = jnp.pad(edge_weight, (0, pad)).reshape(NS, TCHUNK, K)
    degp = _deg_kernel(col.reshape(NW, NCHUNK, K), ew2.reshape(NW, NCHUNK, K)).T
    y1 = _mm_scale(x, W1, degp)
    s1 = _msg_kernel(y1, row, col, ew2)
    y2 = _combine(s1, degp, y1, b1, W2)
    s2 = _msg_kernel(y2, row, col, ew2)
    return _final(s2, degp, y2, b2, batch_idx, L1W, L1b, ln_g, ln_b, L2W, L2b)
